# baseline (speedup 1.0000x reference)
import functools

import jax
import jax.numpy as jnp
from jax.experimental import pallas as pl
from jax.experimental.pallas import tpu as pltpu
from jax.experimental.shard_map import shard_map
from jax.sharding import PartitionSpec as P

_N = 10000


def _mm_kernel(a_ref, w_ref, o_ref):
    a = a_ref[...].astype(jnp.bfloat16)
    w = w_ref[...].astype(jnp.bfloat16)
    o_ref[...] = jnp.dot(a, w, preferred_element_type=jnp.float32).astype(
        jnp.bfloat16)


def _agg_kernel(adj_ref, s_ref, b_ref, o_ref, *, relu):
    adj_blk = adj_ref[...].astype(jnp.bfloat16)
    acc = jnp.dot(adj_blk, s_ref[...], preferred_element_type=jnp.float32)
    acc = acc + b_ref[...]
    if relu:
        acc = jnp.maximum(acc, 0.0)
        o_ref[...] = acc.astype(jnp.bfloat16)
    else:
        o_ref[...] = acc


def _partial_lsm_kernel(h_ref, m_ref, se_ref):
    h = h_ref[...]
    m = jnp.max(h, axis=0, keepdims=True)
    m_ref[...] = m
    se_ref[...] = jnp.sum(jnp.exp(h - m), axis=0, keepdims=True)


def _sub_kernel(h_ref, lse_ref, o_ref):
    o_ref[...] = h_ref[...] - lse_ref[...]


def _mm(a, w):
    m, _ = a.shape
    _, n = w.shape
    return pl.pallas_call(
        _mm_kernel,
        out_shape=jax.ShapeDtypeStruct((m, n), jnp.bfloat16),
    )(a, w)


def _agg(adj, s, b, relu, out_dtype, bm):
    n_rows = adj.shape[0]
    f = s.shape[1]
    return pl.pallas_call(
        functools.partial(_agg_kernel, relu=relu),
        grid=(n_rows // bm,),
        in_specs=[
            pl.BlockSpec((bm, _N), lambda i: (i, 0)),
            pl.BlockSpec((_N, f), lambda i: (0, 0)),
            pl.BlockSpec((1, f), lambda i: (0, 0)),
        ],
        out_specs=pl.BlockSpec((bm, f), lambda i: (i, 0)),
        out_shape=jax.ShapeDtypeStruct((n_rows, f), out_dtype),
        compiler_params=pltpu.CompilerParams(
            dimension_semantics=("parallel",)),
    )(adj, s, b)


def _shard_impl(x2d, adj_loc, W0, b0, W1, b1, *, bm):
    f2 = W1.shape[1]
    s1 = _mm(x2d, W0)
    h1_loc = _agg(adj_loc, s1, b0, True, jnp.bfloat16, bm)
    h1 = jax.lax.all_gather(h1_loc, "d", axis=0, tiled=True)
    s2 = _mm(h1, W1)
    h2_loc = _agg(adj_loc, s2, b1, False, jnp.float32, bm)

    m_loc, se_loc = pl.pallas_call(
        _partial_lsm_kernel,
        out_shape=(jax.ShapeDtypeStruct((1, f2), jnp.float32),
                   jax.ShapeDtypeStruct((1, f2), jnp.float32)),
    )(h2_loc)
    m_g = jax.lax.pmax(m_loc, "d")
    se_g = jax.lax.psum(se_loc * jnp.exp(m_loc - m_g), "d")
    lse = jnp.log(se_g) + m_g
    out_loc = pl.pallas_call(
        _sub_kernel,
        out_shape=jax.ShapeDtypeStruct(h2_loc.shape, jnp.float32),
    )(h2_loc, lse)
    return out_loc[None]


def kernel(x, adj, W0, b0, W1, b1):
    nd = 2 if len(jax.devices()) >= 2 else 1
    mesh = jax.make_mesh((nd,), ("d",))
    bm = 400 // nd
    x2d = x.reshape(_N, x.shape[-1])
    ns = jax.sharding.NamedSharding
    adj = jax.reshard(adj, ns(mesh, P("d", None)))
    args = [jax.reshard(a, ns(mesh, P()))
            for a in (x2d, W0, b0.reshape(1, -1), W1, b1.reshape(1, -1))]
    f = shard_map(
        functools.partial(_shard_impl, bm=bm),
        mesh=mesh,
        in_specs=(P(), P("d", None), P(), P(), P(), P()),
        out_specs=P(None, "d", None),
        check_rep=False,
    )
    out = f(args[0], adj, args[1], args[2], args[3], args[4])
    return out


# speedup vs baseline: 2.6560x; 2.6560x over previous
import numpy as np

import jax
import jax.numpy as jnp
from jax.experimental import pallas as pl
from jax.experimental.pallas import tpu as pltpu

_N = 10000
_BM = 1000
_BK = 1024
_RB = _N // _BM
_CB = -(-_N // _BK)
_NPAD = _CB * _BK


def _dual(r, c):
    return _BK * (c + 1) <= _BM * r


def _make_schedule():
    rs, cs, ph = [], [], []
    for r in range(_RB):
        for c in range(_CB):
            rs.append(r)
            cs.append(c)
            ph.append(0)
    for r in range(_RB):
        for c in range(_CB):
            if not _dual(r, c):
                rs.append(r)
                cs.append(c)
                ph.append(1)
    return (np.asarray(rs, np.int32), np.asarray(cs, np.int32),
            np.asarray(ph, np.int32))


_RTAB, _CTAB, _PTAB = _make_schedule()
_NSTEPS = _RTAB.shape[0]


def _mm_kernel(a_ref, w_ref, o_ref):
    a = a_ref[...].astype(jnp.bfloat16)
    w = w_ref[...].astype(jnp.bfloat16)
    o_ref[...] = jnp.dot(a, w, preferred_element_type=jnp.float32).astype(
        jnp.bfloat16)


def _lsm_kernel(h_ref, o_ref):
    h = h_ref[...]
    m = jnp.max(h, axis=0, keepdims=True)
    lse = jnp.log(jnp.sum(jnp.exp(h - m), axis=0, keepdims=True)) + m
    o_ref[...] = h - lse


def _fused_kernel(rtab_ref, ctab_ref, ptab_ref, adj_ref, s1_ref, b0_ref,
                  w1_ref, out_ref, h1p_ref, s2_ref):
    t = pl.program_id(0)
    r = rtab_ref[t]
    c = ctab_ref[t]
    ph = ptab_ref[t]

    @pl.when(t == 0)
    def _init_s2_pad():
        s2_ref[pl.ds(_N, _NPAD - _N), :] = jnp.zeros(
            (_NPAD - _N, s2_ref.shape[1]), jnp.bfloat16)

    col_ids = jax.lax.broadcasted_iota(jnp.int32, (_BM, _BK), 1)
    valid = _N - c * _BK
    tile = jnp.where(col_ids < valid, adj_ref[...], 0.0).astype(jnp.bfloat16)

    @pl.when(ph == 0)
    def _pass1():
        part = jnp.dot(tile, s1_ref[pl.ds(c * _BK, _BK), :],
                       preferred_element_type=jnp.float32)

        @pl.when(c == 0)
        def _():
            h1p_ref[...] = part

        @pl.when(c != 0)
        def _():
            h1p_ref[...] = h1p_ref[...] + part

        @pl.when(c == _CB - 1)
        def _finalize_stripe():
            h1 = jnp.maximum(h1p_ref[...] + b0_ref[...], 0.0)
            s2_blk = jnp.dot(h1.astype(jnp.bfloat16),
                             w1_ref[...].astype(jnp.bfloat16),
                             preferred_element_type=jnp.float32)
            s2_ref[pl.ds(r * _BM, _BM), :] = s2_blk.astype(jnp.bfloat16)

    @pl.when(jnp.logical_or(ph == 1, _BK * (c + 1) <= _BM * r))
    def _pass2():
        contrib = jnp.dot(tile, s2_ref[pl.ds(c * _BK, _BK), :],
                          preferred_element_type=jnp.float32)

        @pl.when(c == 0)
        def _():
            out_ref[pl.ds(r * _BM, _BM), :] = contrib

        @pl.when(c != 0)
        def _():
            out_ref[pl.ds(r * _BM, _BM), :] = (
                out_ref[pl.ds(r * _BM, _BM), :] + contrib)


def kernel(x, adj, W0, b0, W1, b1):
    f1 = W0.shape[1]
    f2 = W1.shape[1]
    x2d = x.reshape(_N, x.shape[-1])
    x_pad = jnp.pad(x2d, ((0, _NPAD - _N), (0, 0)))
    s1 = pl.pallas_call(
        _mm_kernel,
        out_shape=jax.ShapeDtypeStruct((_NPAD, f1), jnp.bfloat16),
    )(x_pad, W0)

    h2 = pl.pallas_call(
        _fused_kernel,
        grid_spec=pltpu.PrefetchScalarGridSpec(
            num_scalar_prefetch=3,
            grid=(_NSTEPS,),
            in_specs=[
                pl.BlockSpec((_BM, _BK),
                             lambda t, rt, ct, pt: (rt[t], ct[t])),
                pl.BlockSpec((_NPAD, f1), lambda t, rt, ct, pt: (0, 0)),
                pl.BlockSpec((1, f1), lambda t, rt, ct, pt: (0, 0)),
                pl.BlockSpec((f1, f2), lambda t, rt, ct, pt: (0, 0)),
            ],
            out_specs=pl.BlockSpec((_N, f2), lambda t, rt, ct, pt: (0, 0)),
            scratch_shapes=[
                pltpu.VMEM((_BM, f1), jnp.float32),
                pltpu.VMEM((_NPAD, f2), jnp.bfloat16),
            ],
        ),
        out_shape=jax.ShapeDtypeStruct((_N, f2), jnp.float32),
        compiler_params=pltpu.CompilerParams(
            dimension_semantics=("arbitrary",)),
    )(jnp.asarray(_RTAB), jnp.asarray(_CTAB), jnp.asarray(_PTAB),
      adj, s1, b0.reshape(1, -1), W1)

    out = pl.pallas_call(
        _lsm_kernel,
        out_shape=jax.ShapeDtypeStruct((_N, f2), jnp.float32),
    )(h2)
    return out.reshape(1, _N, f2)


# speedup vs baseline: 2.9255x; 1.1015x over previous
import numpy as np

import jax
import jax.numpy as jnp
from jax.experimental import pallas as pl
from jax.experimental.pallas import tpu as pltpu

_N = 10000
_BM = 1000
_BK = 1024
_RB = _N // _BM
_CB = -(-_N // _BK)
_NPAD = _CB * _BK
_F1 = 128
_F2 = 64


def _dual(r, c):
    return _BK * (c + 1) <= _BM * r


def _make_schedule():
    rs, cs, ph = [], [], []
    for r in range(_RB):
        for c in range(_CB):
            rs.append(r)
            cs.append(c)
            ph.append(0)
    for r in range(_RB):
        for c in range(_CB):
            if not _dual(r, c):
                rs.append(r)
                cs.append(c)
                ph.append(1)
    return (np.asarray(rs, np.int32), np.asarray(cs, np.int32),
            np.asarray(ph, np.int32))


_RTAB, _CTAB, _PTAB = _make_schedule()
_NSTEPS = _RTAB.shape[0]


def _mm_kernel(a_ref, w_ref, o_ref):
    a = a_ref[...].astype(jnp.bfloat16)
    w = w_ref[...].astype(jnp.bfloat16)
    o_ref[...] = jnp.dot(a, w, preferred_element_type=jnp.float32)


def _lsm_kernel(h_ref, o_ref):
    h = h_ref[...]
    m = jnp.max(h, axis=0, keepdims=True)
    lse = jnp.log(jnp.sum(jnp.exp(h - m), axis=0, keepdims=True)) + m
    o_ref[...] = h - lse


def _fused_kernel(rtab_ref, ctab_ref, ptab_ref, adj_ref, s1_ref, b0_ref,
                  w1_ref, out_ref, h1p_ref, s_ref):
    t = pl.program_id(0)
    r = rtab_ref[t]
    c = ctab_ref[t]
    ph = ptab_ref[t]

    @pl.when(t == 0)
    def _init_s():
        s_ref[:, :_F1] = s1_ref[...]
        s_ref[:, _F1:] = jnp.zeros((_NPAD, _F2), jnp.float32)

    tile = adj_ref[...].astype(jnp.bfloat16)
    s_blk = s_ref[pl.ds(c * _BK, _BK), :].astype(jnp.bfloat16)
    res = jnp.dot(tile, s_blk, preferred_element_type=jnp.float32)

    @pl.when(ph == 0)
    def _pass1():
        part = res[:, :_F1]

        @pl.when(c == 0)
        def _():
            h1p_ref[...] = part

        @pl.when(c != 0)
        def _():
            h1p_ref[...] = h1p_ref[...] + part

        @pl.when(c == _CB - 1)
        def _finalize_stripe():
            h1 = jnp.maximum(h1p_ref[...] + b0_ref[...], 0.0)
            s2_blk = jnp.dot(h1.astype(jnp.bfloat16),
                             w1_ref[...].astype(jnp.bfloat16),
                             preferred_element_type=jnp.float32)
            s_ref[pl.ds(r * _BM, _BM), _F1:] = s2_blk

    @pl.when(jnp.logical_or(ph == 1, _BK * (c + 1) <= _BM * r))
    def _pass2():
        contrib = res[:, _F1:]

        @pl.when(c == 0)
        def _():
            out_ref[pl.ds(r * _BM, _BM), :] = contrib

        @pl.when(c != 0)
        def _():
            out_ref[pl.ds(r * _BM, _BM), :] = (
                out_ref[pl.ds(r * _BM, _BM), :] + contrib)


def kernel(x, adj, W0, b0, W1, b1):
    x2d = x.reshape(_N, x.shape[-1])
    x_pad = jnp.pad(x2d, ((0, _NPAD - _N), (0, 0)))
    s1 = pl.pallas_call(
        _mm_kernel,
        out_shape=jax.ShapeDtypeStruct((_NPAD, _F1), jnp.float32),
    )(x_pad, W0)

    h2 = pl.pallas_call(
        _fused_kernel,
        grid_spec=pltpu.PrefetchScalarGridSpec(
            num_scalar_prefetch=3,
            grid=(_NSTEPS,),
            in_specs=[
                pl.BlockSpec((_BM, _BK),
                             lambda t, rt, ct, pt: (rt[t], ct[t])),
                pl.BlockSpec((_NPAD, _F1), lambda t, rt, ct, pt: (0, 0)),
                pl.BlockSpec((1, _F1), lambda t, rt, ct, pt: (0, 0)),
                pl.BlockSpec((_F1, _F2), lambda t, rt, ct, pt: (0, 0)),
            ],
            out_specs=pl.BlockSpec((_N, _F2), lambda t, rt, ct, pt: (0, 0)),
            scratch_shapes=[
                pltpu.VMEM((_BM, _F1), jnp.float32),
                pltpu.VMEM((_NPAD, _F1 + _F2), jnp.float32),
            ],
        ),
        out_shape=jax.ShapeDtypeStruct((_N, _F2), jnp.float32),
        compiler_params=pltpu.CompilerParams(
            dimension_semantics=("arbitrary",)),
    )(jnp.asarray(_RTAB), jnp.asarray(_CTAB), jnp.asarray(_PTAB),
      adj, s1, b0.reshape(1, -1), W1)

    out = pl.pallas_call(
        _lsm_kernel,
        out_shape=jax.ShapeDtypeStruct((_N, _F2), jnp.float32),
    )(h2)
    return out.reshape(1, _N, _F2)
